# baseline (speedup 1.0000x reference)
import jax
import jax.numpy as jnp
from jax import lax
from jax.experimental import pallas as pl
from jax.experimental.pallas import tpu as pltpu
from jax.experimental.pallas import tpu_sc as plsc

BATCH = 4096
HIST = 50
D_DIM = 64
I_DIM = 1000000

NC = 2
NS = 16
NW = NC * NS
BBLK = BATCH // NW
NGRP = HIST
NBUF = 2
N_OUTER = NGRP // NBUF
DSUB = D_DIM // 8


def _emb_body(xt_hbm, table_hbm, out_hbm, idx_v, rows_a, rows_b, tile_a, tile_b,
              gsem, osem):
    rows_bufs = (rows_a, rows_b)
    tile_bufs = (tile_a, tile_b)
    cid = lax.axis_index("c")
    sid = lax.axis_index("s")
    wid = sid * NC + cid

    pltpu.sync_copy(xt_hbm.at[:, pl.ds(wid * BBLK, BBLK)], idx_v)

    def gather_start(h, b):
        pltpu.make_async_copy(
            table_hbm.at[idx_v.at[h]], rows_bufs[b], gsem.at[b]
        ).start()

    def gather_wait(b):
        pltpu.make_async_copy(
            table_hbm.at[idx_v.at[0]], rows_bufs[b], gsem.at[b]
        ).wait()

    def out_start(h, b):
        pltpu.make_async_copy(
            tile_bufs[b], out_hbm.at[h, :, wid], osem.at[b]
        ).start()

    def out_wait(b):
        pltpu.make_async_copy(
            tile_bufs[b], out_hbm.at[0, :, 0], osem.at[b]
        ).wait()

    iota16 = lax.iota(jnp.int32, 16)

    def transpose_group(b):
        def r_body(r, _):
            for s in range(8):
                colidx = jnp.zeros((16,), jnp.int32) + (r * 8 + s)
                for l0 in range(8):
                    rowidx = iota16 + (l0 * 16)
                    v = plsc.load_gather(rows_bufs[b], [rowidx, colidx])
                    tile_bufs[b][r, s, pl.ds(l0 * 16, 16)] = v
            return ()

        lax.fori_loop(0, DSUB, r_body, ())

    for b in range(NBUF):
        gather_start(b, b)

    def outer(g, _):
        for b in range(NBUF):
            h = g * NBUF + b
            gather_wait(b)

            @pl.when(h >= NBUF)
            def _():
                out_wait(b)

            transpose_group(b)
            out_start(h, b)

            @pl.when(h + NBUF < NGRP)
            def _():
                gather_start(h + NBUF, b)

        return ()

    lax.fori_loop(0, N_OUTER, outer, ())

    for b in range(NBUF):
        out_wait(b)


def kernel(x, item_emb_mat):
    xt = jnp.transpose(x).astype(jnp.int32)
    tab = item_emb_mat[:I_DIM]
    mesh = plsc.VectorSubcoreMesh(core_axis_name="c", subcore_axis_name="s")
    out5 = pl.kernel(
        _emb_body,
        out_type=jax.ShapeDtypeStruct((HIST, DSUB, NW, 8, 128), jnp.float32),
        mesh=mesh,
        compiler_params=pltpu.CompilerParams(
            use_tc_tiling_on_sc=False, needs_layout_passes=False
        ),
        scratch_types=[
            pltpu.VMEM((NGRP, BBLK), jnp.int32),
            pltpu.VMEM((BBLK, D_DIM), jnp.float32),
            pltpu.VMEM((BBLK, D_DIM), jnp.float32),
            pltpu.VMEM((DSUB, 8, 128), jnp.float32),
            pltpu.VMEM((DSUB, 8, 128), jnp.float32),
            pltpu.SemaphoreType.DMA((NBUF,)),
            pltpu.SemaphoreType.DMA((NBUF,)),
        ],
    )(xt, tab)
    return out5.transpose((2, 4, 0, 1, 3)).reshape(BATCH, HIST, D_DIM)


# speedup vs baseline: 1.9668x; 1.9668x over previous
import jax
import jax.numpy as jnp
from jax import lax
from jax.experimental import pallas as pl
from jax.experimental.pallas import tpu as pltpu
from jax.experimental.pallas import tpu_sc as plsc

BATCH = 4096
HIST = 50
D_DIM = 64
I_DIM = 1000000

NC = 2
NS = 16
NW = NC * NS
B_TOTAL = BATCH * HIST
PER_W = B_TOTAL // NW
CHUNK = 128
NCHUNK = PER_W // CHUNK
NBUF = 2
N_OUTER = NCHUNK // NBUF


def _gather_body(x_hbm, tab_hbm, out_hbm, idx_v, rows_a, rows_b, gsem, osem):
    cid = lax.axis_index("c")
    sid = lax.axis_index("s")
    wid = sid * NC + cid
    rows_bufs = (rows_a, rows_b)

    pltpu.sync_copy(x_hbm.at[pl.ds(wid * PER_W, PER_W)], idx_v)

    def gather_start(gid, b):
        def row16(k16, _):
            v = idx_v[pl.ds((gid * CHUNK + k16 * 16), 16)]
            for j in range(16):
                i = v[j]
                pltpu.make_async_copy(
                    tab_hbm.at[i >> 3, i & 7],
                    rows_bufs[b].at[k16 * 16 + j],
                    gsem.at[b],
                ).start()
            return ()

        lax.fori_loop(0, CHUNK // 16, row16, ())

    def gather_wait(b):
        def row(k, _):
            pltpu.make_async_copy(
                tab_hbm.at[0, 0], rows_bufs[b].at[0], gsem.at[b]
            ).wait()
            return ()

        lax.fori_loop(0, CHUNK, row, ())

    def out_start(gid, b):
        pltpu.make_async_copy(
            rows_bufs[b], out_hbm.at[wid * NCHUNK + gid], osem.at[b]
        ).start()

    def out_wait(b):
        pltpu.make_async_copy(
            rows_bufs[b], out_hbm.at[0], osem.at[b]
        ).wait()

    for b in range(NBUF):
        gather_start(b, b)

    def outer(g, _):
        for b in range(NBUF):
            gid = g * NBUF + b
            gather_wait(b)
            out_start(gid, b)
            out_wait(b)

            @pl.when(gid + NBUF < NCHUNK)
            def _():
                gather_start(gid + NBUF, b)

        return ()

    lax.fori_loop(0, N_OUTER, outer, ())


def kernel(x, item_emb_mat):
    tab3 = item_emb_mat[:I_DIM].reshape(I_DIM // 8, 8, D_DIM)
    x_flat = x.reshape(B_TOTAL).astype(jnp.int32)
    mesh = plsc.VectorSubcoreMesh(core_axis_name="c", subcore_axis_name="s")
    out = pl.kernel(
        _gather_body,
        out_type=jax.ShapeDtypeStruct((NW * NCHUNK, CHUNK, D_DIM), jnp.float32),
        mesh=mesh,
        compiler_params=pltpu.CompilerParams(use_tc_tiling_on_sc=True),
        scratch_types=[
            pltpu.VMEM((PER_W,), jnp.int32),
            pltpu.VMEM((CHUNK, D_DIM), jnp.float32),
            pltpu.VMEM((CHUNK, D_DIM), jnp.float32),
            pltpu.SemaphoreType.DMA((NBUF,)),
            pltpu.SemaphoreType.DMA((NBUF,)),
        ],
    )(x_flat, tab3)
    return out.reshape(BATCH, HIST, D_DIM)


# speedup vs baseline: 2.2431x; 1.1405x over previous
import jax
import jax.numpy as jnp
from jax import lax
from jax.experimental import pallas as pl
from jax.experimental.pallas import tpu as pltpu
from jax.experimental.pallas import tpu_sc as plsc

BATCH = 4096
HIST = 50
D_DIM = 64
I_DIM = 1000000
HPAD = 64

NC = 2
NS = 16
NW = NC * NS
B_W = BATCH // NW
NBUF = 2
N_OUTER = B_W // NBUF


def _gather_body(x_hbm, tab_hbm, out_hbm, idx_v, rows_a, rows_b, gsem, osem):
    cid = lax.axis_index("c")
    sid = lax.axis_index("s")
    wid = sid * NC + cid
    rows_bufs = (rows_a, rows_b)

    pltpu.sync_copy(x_hbm.at[pl.ds(wid * B_W, B_W), :], idx_v)

    def gather_start(bb, b):
        for k16 in range(HIST // 16 + 1):
            v = idx_v[bb, pl.ds(k16 * 16, 16)]
            for j in range(16):
                k = k16 * 16 + j
                if k < HIST:
                    i = v[j]
                    pltpu.make_async_copy(
                        tab_hbm.at[i >> 3, i & 7], rows_bufs[b].at[k], gsem.at[b]
                    ).start()

    def gather_wait(b):
        def row(k, _):
            pltpu.make_async_copy(
                tab_hbm.at[0, 0], rows_bufs[b].at[0], gsem.at[b]
            ).wait()
            return ()

        lax.fori_loop(0, HIST, row, ())

    def out_start(bb, b):
        pltpu.make_async_copy(
            rows_bufs[b], out_hbm.at[wid * B_W + bb], osem.at[b]
        ).start()

    def out_wait(b):
        pltpu.make_async_copy(
            rows_bufs[b], out_hbm.at[0], osem.at[b]
        ).wait()

    for b in range(NBUF):
        gather_start(b, b)

    def outer(g, _):
        for b in range(NBUF):
            bb = g * NBUF + b
            gather_wait(b)
            out_start(bb, b)
            out_wait(b)

            @pl.when(bb + NBUF < B_W)
            def _():
                gather_start(bb + NBUF, b)

        return ()

    lax.fori_loop(0, N_OUTER, outer, ())


def kernel(x, item_emb_mat):
    tab3 = item_emb_mat[:I_DIM].reshape(I_DIM // 8, 8, D_DIM)
    xpad = jnp.pad(x.astype(jnp.int32), ((0, 0), (0, HPAD - HIST)))
    mesh = plsc.VectorSubcoreMesh(core_axis_name="c", subcore_axis_name="s")
    out = pl.kernel(
        _gather_body,
        out_type=jax.ShapeDtypeStruct((BATCH, HIST, D_DIM), jnp.float32),
        mesh=mesh,
        compiler_params=pltpu.CompilerParams(use_tc_tiling_on_sc=True),
        scratch_types=[
            pltpu.VMEM((B_W, HPAD), jnp.int32),
            pltpu.VMEM((HIST, D_DIM), jnp.float32),
            pltpu.VMEM((HIST, D_DIM), jnp.float32),
            pltpu.SemaphoreType.DMA((NBUF,)),
            pltpu.SemaphoreType.DMA((NBUF,)),
        ],
    )(xpad, tab3)
    return out


# speedup vs baseline: 2.2889x; 1.0204x over previous
import jax
import jax.numpy as jnp
from jax import lax
from jax.experimental import pallas as pl
from jax.experimental.pallas import tpu as pltpu
from jax.experimental.pallas import tpu_sc as plsc

BATCH = 4096
HIST = 50
D_DIM = 64
I_DIM = 1000000
HPAD = 64

NC = 2
NS = 16
NW = NC * NS
B_W = BATCH // NW
NBUF = 4
N_OUTER = B_W // NBUF


def _gather_body(x_hbm, tab_hbm, out_hbm, idx_v, rows_a, rows_b, rows_c, rows_d,
                 gsem, osem):
    cid = lax.axis_index("c")
    sid = lax.axis_index("s")
    wid = sid * NC + cid
    rows_bufs = (rows_a, rows_b, rows_c, rows_d)

    pltpu.sync_copy(x_hbm.at[pl.ds(wid * B_W, B_W), :], idx_v)

    def gather_start(bb, b):
        for k16 in range(HIST // 16 + 1):
            v = idx_v[bb, pl.ds(k16 * 16, 16)]
            for j in range(16):
                k = k16 * 16 + j
                if k < HIST:
                    i = v[j]
                    pltpu.make_async_copy(
                        tab_hbm.at[i >> 3, i & 7], rows_bufs[b].at[k], gsem.at[b]
                    ).start()

    def gather_wait(b):
        def row(k, _):
            pltpu.make_async_copy(
                tab_hbm.at[0, 0], rows_bufs[b].at[0], gsem.at[b]
            ).wait()
            return ()

        lax.fori_loop(0, HIST, row, ())

    def out_start(bb, b):
        pltpu.make_async_copy(
            rows_bufs[b], out_hbm.at[wid * B_W + bb], osem.at[b]
        ).start()

    def out_wait(b):
        pltpu.make_async_copy(
            rows_bufs[b], out_hbm.at[0], osem.at[b]
        ).wait()

    for b in range(NBUF):
        gather_start(b, b)

    def outer(g, _):
        for b in range(NBUF):
            bb = g * NBUF + b
            gather_wait(b)
            out_start(bb, b)
            out_wait(b)

            @pl.when(bb + NBUF < B_W)
            def _():
                gather_start(bb + NBUF, b)

        return ()

    lax.fori_loop(0, N_OUTER, outer, ())


def kernel(x, item_emb_mat):
    tab3 = item_emb_mat[:I_DIM].reshape(I_DIM // 8, 8, D_DIM)
    xpad = jnp.pad(x.astype(jnp.int32), ((0, 0), (0, HPAD - HIST)))
    mesh = plsc.VectorSubcoreMesh(core_axis_name="c", subcore_axis_name="s")
    out = pl.kernel(
        _gather_body,
        out_type=jax.ShapeDtypeStruct((BATCH, HIST, D_DIM), jnp.float32),
        mesh=mesh,
        compiler_params=pltpu.CompilerParams(use_tc_tiling_on_sc=True),
        scratch_types=[
            pltpu.VMEM((B_W, HPAD), jnp.int32),
            pltpu.VMEM((HIST, D_DIM), jnp.float32),
            pltpu.VMEM((HIST, D_DIM), jnp.float32),
            pltpu.VMEM((HIST, D_DIM), jnp.float32),
            pltpu.VMEM((HIST, D_DIM), jnp.float32),
            pltpu.SemaphoreType.DMA((NBUF,)),
            pltpu.SemaphoreType.DMA((NBUF,)),
        ],
    )(xpad, tab3)
    return out
